# baseline (speedup 1.0000x reference)
import jax
import jax.numpy as jnp
from jax.experimental import pallas as pl
from jax.experimental.pallas import tpu as pltpu

BLOCK_M = 512


def _router_block(h_ref, w_ref, out_ref):
    h = h_ref[...]
    w = w_ref[...]
    logits = jax.lax.dot_general(
        h, w, (((1,), (0,)), ((), ())), preferred_element_type=jnp.float32
    )
    m = jnp.max(logits, axis=-1, keepdims=True)
    e = jnp.exp(logits - m)
    out_ref[...] = e / jnp.sum(e, axis=-1, keepdims=True)


def kernel(hidden_states, gate_weight):
    n_tokens, hidden = hidden_states.shape
    n_experts = gate_weight.shape[0]
    w_t = gate_weight.T
    grid = (n_tokens // BLOCK_M,)
    return pl.pallas_call(
        _router_block,
        grid=grid,
        in_specs=[
            pl.BlockSpec((BLOCK_M, hidden), lambda i: (i, 0)),
            pl.BlockSpec((hidden, n_experts), lambda i: (0, 0)),
        ],
        out_specs=pl.BlockSpec((BLOCK_M, n_experts), lambda i: (i, 0)),
        out_shape=jax.ShapeDtypeStruct((n_tokens, n_experts), jnp.float32),
        compiler_params=pltpu.CompilerParams(
            dimension_semantics=("parallel",),
        ),
    )(hidden_states, w_t)


# speedup vs baseline: 1.0333x; 1.0333x over previous
import jax
import jax.numpy as jnp
from jax.experimental import pallas as pl
from jax.experimental.pallas import tpu as pltpu

BLOCK_M = 512


def _router_block(h_ref, w_ref, out_ref):
    h = h_ref[...]
    w = w_ref[...]
    logits = jax.lax.dot_general(
        h, w, (((1,), (1,)), ((), ())), preferred_element_type=jnp.float32
    )
    m = jnp.max(logits, axis=-1, keepdims=True)
    e = jnp.exp(logits - m)
    out_ref[...] = e / jnp.sum(e, axis=-1, keepdims=True)


def kernel(hidden_states, gate_weight):
    n_tokens, hidden = hidden_states.shape
    n_experts = gate_weight.shape[0]
    grid = (n_tokens // BLOCK_M,)
    return pl.pallas_call(
        _router_block,
        grid=grid,
        in_specs=[
            pl.BlockSpec((BLOCK_M, hidden), lambda i: (i, 0)),
            pl.BlockSpec((n_experts, hidden), lambda i: (0, 0)),
        ],
        out_specs=pl.BlockSpec((BLOCK_M, n_experts), lambda i: (i, 0)),
        out_shape=jax.ShapeDtypeStruct((n_tokens, n_experts), jnp.float32),
        compiler_params=pltpu.CompilerParams(
            dimension_semantics=("parallel",),
        ),
    )(hidden_states, gate_weight)


# speedup vs baseline: 1.0340x; 1.0006x over previous
import jax
import jax.numpy as jnp
from jax.experimental import pallas as pl
from jax.experimental.pallas import tpu as pltpu

BLOCK_M = 512
SPLITS = 2


def _router_block(*refs):
    h_refs = refs[:SPLITS]
    w_ref = refs[SPLITS]
    out_ref = refs[SPLITS + 1]
    hidden = w_ref.shape[1]
    chunk = hidden // SPLITS
    logits = None
    for s in range(SPLITS):
        w_s = w_ref[:, s * chunk:(s + 1) * chunk]
        part = jax.lax.dot_general(
            h_refs[s][...], w_s, (((1,), (1,)), ((), ())),
            preferred_element_type=jnp.float32,
        )
        logits = part if logits is None else logits + part
    m = jnp.max(logits, axis=-1, keepdims=True)
    e = jnp.exp(logits - m)
    out_ref[...] = e / jnp.sum(e, axis=-1, keepdims=True)


def kernel(hidden_states, gate_weight):
    n_tokens, hidden = hidden_states.shape
    n_experts = gate_weight.shape[0]
    chunk = hidden // SPLITS
    grid = (n_tokens // BLOCK_M,)
    h_specs = [
        pl.BlockSpec((BLOCK_M, chunk), lambda i, s=s: (i, s)) for s in range(SPLITS)
    ]
    return pl.pallas_call(
        _router_block,
        grid=grid,
        in_specs=h_specs + [pl.BlockSpec((n_experts, hidden), lambda i: (0, 0))],
        out_specs=pl.BlockSpec((BLOCK_M, n_experts), lambda i: (i, 0)),
        out_shape=jax.ShapeDtypeStruct((n_tokens, n_experts), jnp.float32),
        compiler_params=pltpu.CompilerParams(
            dimension_semantics=("arbitrary",),
        ),
    )(*([hidden_states] * SPLITS), gate_weight)


# speedup vs baseline: 1.0382x; 1.0041x over previous
import jax
import jax.numpy as jnp
from jax.experimental import pallas as pl
from jax.experimental.pallas import tpu as pltpu

BLOCK_M = 1024
SPLITS = 1


def _router_block(*refs):
    h_refs = refs[:SPLITS]
    w_ref = refs[SPLITS]
    out_ref = refs[SPLITS + 1]
    hidden = w_ref.shape[1]
    chunk = hidden // SPLITS
    logits = None
    for s in range(SPLITS):
        w_s = w_ref[:, s * chunk:(s + 1) * chunk]
        part = jax.lax.dot_general(
            h_refs[s][...], w_s, (((1,), (1,)), ((), ())),
            preferred_element_type=jnp.float32,
        )
        logits = part if logits is None else logits + part
    m = jnp.max(logits, axis=-1, keepdims=True)
    e = jnp.exp(logits - m)
    out_ref[...] = e / jnp.sum(e, axis=-1, keepdims=True)


def kernel(hidden_states, gate_weight):
    n_tokens, hidden = hidden_states.shape
    n_experts = gate_weight.shape[0]
    chunk = hidden // SPLITS
    grid = (n_tokens // BLOCK_M,)
    h_specs = [
        pl.BlockSpec((BLOCK_M, chunk), lambda i, s=s: (i, s)) for s in range(SPLITS)
    ]
    return pl.pallas_call(
        _router_block,
        grid=grid,
        in_specs=h_specs + [pl.BlockSpec((n_experts, hidden), lambda i: (0, 0))],
        out_specs=pl.BlockSpec((BLOCK_M, n_experts), lambda i: (i, 0)),
        out_shape=jax.ShapeDtypeStruct((n_tokens, n_experts), jnp.float32),
        compiler_params=pltpu.CompilerParams(
            dimension_semantics=("arbitrary",),
        ),
    )(*([hidden_states] * SPLITS), gate_weight)
